# baseline (speedup 1.0000x reference)
import jax
import jax.numpy as jnp
from jax import lax
from jax.experimental import pallas as pl
from jax.experimental.pallas import tpu as pltpu
from jax.experimental.pallas import tpu_sc as plsc

N = 1_000_000
C = 32
HALF = 16
OUT_SIZE = 64
S = OUT_SIZE ** 3
SCALE = 0.125

NS = 16
L = 16

K = 256
ROWS_MAIN = 62_464
CHUNKS_MAIN = ROWS_MAIN // K
ROWS_LAST = N - (NS - 1) * ROWS_MAIN
TAIL = ROWS_LAST - 246 * K

DUMP = 4352
RSIZE = 98_304
NSWEEP = 3


def _body(feat, sg, out,
          sgb, fb, ib, ibuf_t, obuf, zbuf, spmem,
          gs0, gs1, gs2, gs3, ss0, ss1, ss2, ss3, osem):
    core = lax.axis_index("c")
    tile = lax.axis_index("s")
    base_row = tile * ROWS_MAIN
    col0 = core * HALF
    iota = lax.iota(jnp.int32, L)
    zeros16 = jnp.zeros((L,), jnp.float32)
    gsem = (gs0, gs1, gs2, gs3)
    ssem = (ss0, ss1, ss2, ss3)

    def _zb(t, _):
        zbuf[t, :] = zeros16
        return _
    lax.fori_loop(0, 64, _zb, None)

    def fire_gather(s, c):
        hoff = base_row + c * K
        pltpu.async_copy(sg.at[pl.ds(hoff, K)], sgb.at[s], gsem[s])
        pltpu.async_copy(feat.at[pl.ds(core * N + hoff, K), :],
                         fb.at[s], gsem[s])

    def wait_gather(s):
        pltpu.make_async_copy(sg.at[pl.ds(0, K)], sgb.at[s], gsem[s]).wait()
        pltpu.make_async_copy(feat.at[pl.ds(0, K), :],
                              fb.at[s], gsem[s]).wait()

    def fire_scatter(s):
        pltpu.async_copy(fb.at[s, pl.ds(0, 128), :],
                         spmem.at[ib.at[2 * s]], ssem[s], add=True)
        pltpu.async_copy(fb.at[s, pl.ds(128, 128), :],
                         spmem.at[ib.at[2 * s + 1]], ssem[s], add=True)

    def wait_scatter(s):
        pltpu.make_async_copy(fb.at[s], spmem.at[ib.at[2 * s]],
                              ssem[s]).wait()

    def build_idx(s, rbase):
        for q in range(0, K, L):
            seg = sgb[s, pl.ds(q, L)]
            d = seg - rbase
            ok = (d >= 0) & (d < RSIZE)
            dump = (tile * 256 + (q % 256)) + iota
            idx = jnp.where(ok, d + DUMP, dump)
            ib[2 * s + q // 128, pl.ds(q % 128, L)] = idx

    def sweep(p, _):
        rbase = p * RSIZE
        last = p == NSWEEP - 1
        sl = jnp.where(last, 4096, 6144)
        my0 = DUMP + tile * sl

        def _zero(k, _2):
            pltpu.sync_copy(zbuf, spmem.at[pl.ds(my0 + k * 64, 64), :])
            return _2
        lax.fori_loop(0, jnp.where(last, 64, 96), _zero, None)
        plsc.subcore_barrier()

        fire_gather(0, 0)
        fire_gather(1, 1)

        def quad(i, _2):
            for s in range(4):
                c = 4 * i + s
                wait_gather(s)
                build_idx(s, rbase)
                fire_scatter(s)
                t = (s + 2) % 4
                if s < 2:
                    @pl.when(i > 0)
                    def _w():
                        wait_scatter(t)
                else:
                    wait_scatter(t)
                fire_gather(t, c + 2)
            return _2
        lax.fori_loop(0, CHUNKS_MAIN // 4, quad, None)

        wait_gather(0)
        wait_gather(1)
        wait_scatter(2)
        wait_scatter(3)

        @pl.when(tile == NS - 1)
        def _extra():
            build_idx(0, rbase)
            fire_scatter(0)
            build_idx(1, rbase)
            fire_scatter(1)
            wait_scatter(0)
            wait_scatter(1)
            hoff = base_row + 246 * K
            pltpu.async_copy(sg.at[pl.ds(hoff, TAIL)],
                             sgb.at[2, pl.ds(0, TAIL)], gs2)
            pltpu.async_copy(feat.at[pl.ds(core * N + hoff, TAIL), :],
                             fb.at[2, pl.ds(0, TAIL), :], gs2)
            pltpu.make_async_copy(sg.at[pl.ds(0, TAIL)],
                                  sgb.at[2, pl.ds(0, TAIL)], gs2).wait()
            pltpu.make_async_copy(feat.at[pl.ds(0, TAIL), :],
                                  fb.at[2, pl.ds(0, TAIL), :], gs2).wait()
            for q in range(0, TAIL, L):
                seg = sgb[2, pl.ds(q, L)]
                d = seg - rbase
                ok = (d >= 0) & (d < RSIZE)
                dump = (tile * 256 + (q % 256)) + iota
                idx = jnp.where(ok, d + DUMP, dump)
                ibuf_t[pl.ds(q, L)] = idx
            pltpu.async_copy(fb.at[2, pl.ds(0, TAIL), :],
                             spmem.at[ibuf_t], ss2, add=True)
            pltpu.make_async_copy(fb.at[2, pl.ds(0, TAIL), :],
                                  spmem.at[ibuf_t], ss2).wait()

        plsc.subcore_barrier()

        def _copyout(k, _2):
            src0 = my0 + k * 256
            pltpu.sync_copy(spmem.at[pl.ds(src0, 256), :], obuf)

            def _scale(j, _3):
                for t in range(16):
                    row = j * 16 + t
                    obuf[row, :] = obuf[row, :] * SCALE
                return _3
            lax.fori_loop(0, 16, _scale, None)
            orow = rbase + tile * sl + k * 256
            pltpu.async_copy(
                obuf, out.at[pl.ds(orow, 256), pl.ds(col0, HALF)], osem).wait()
            return _2
        lax.fori_loop(0, jnp.where(last, 16, 24), _copyout, None)
        return _

    lax.fori_loop(0, NSWEEP, sweep, None)


def kernel(input_features, coords):
    seg = (((coords[:, 0] >> 1) << 12)
           + ((coords[:, 1] >> 1) << 6)
           + (coords[:, 2] >> 1)).astype(jnp.int32)
    featr = input_features.reshape(N, 2, HALF).transpose(1, 0, 2).reshape(
        2 * N, HALF)
    fn = pl.kernel(
        _body,
        out_type=jax.ShapeDtypeStruct((S, C), jnp.float32),
        mesh=plsc.VectorSubcoreMesh(core_axis_name="c", subcore_axis_name="s"),
        compiler_params=pltpu.CompilerParams(use_tc_tiling_on_sc=False),
        scratch_types=[
            pltpu.VMEM((4, K), jnp.int32),
            pltpu.VMEM((4, K, HALF), jnp.float32),
            pltpu.VMEM((8, 128), jnp.int32),
            pltpu.VMEM((TAIL,), jnp.int32),
            pltpu.VMEM((256, HALF), jnp.float32),
            pltpu.VMEM((64, HALF), jnp.float32),
            pltpu.VMEM_SHARED((DUMP + RSIZE, HALF), jnp.float32),
            pltpu.SemaphoreType.DMA,
            pltpu.SemaphoreType.DMA,
            pltpu.SemaphoreType.DMA,
            pltpu.SemaphoreType.DMA,
            pltpu.SemaphoreType.DMA,
            pltpu.SemaphoreType.DMA,
            pltpu.SemaphoreType.DMA,
            pltpu.SemaphoreType.DMA,
            pltpu.SemaphoreType.DMA,
        ],
    )
    return fn(featr, seg)


# speedup vs baseline: 1.3338x; 1.3338x over previous
import jax
import jax.numpy as jnp
from jax import lax
from jax.experimental import pallas as pl
from jax.experimental.pallas import tpu as pltpu
from jax.experimental.pallas import tpu_sc as plsc

N = 1_000_000
C = 32
HALF = 16
OUT_SIZE = 64
S = OUT_SIZE ** 3
SCALE = 0.125

NS = 16
L = 16

K = 256
ROWS_MAIN = 62_464
CHUNKS_MAIN = ROWS_MAIN // K
ROWS_LAST = N - (NS - 1) * ROWS_MAIN
TAIL = ROWS_LAST - 246 * K

DUMP = 4352
RSIZE = 98_304
NSWEEP = 3


def _body(feat, sg, out,
          sgb, fb, ib, ibuf_t, obuf, zbuf, spmem,
          gs0, gs1, gs2, gs3, ss0, ss1, ss2, ss3, osem):
    core = lax.axis_index("c")
    tile = lax.axis_index("s")
    base_row = tile * ROWS_MAIN
    col0 = core * HALF
    iota = lax.iota(jnp.int32, L)
    zeros16 = jnp.zeros((L,), jnp.float32)
    gsem = (gs0, gs1, gs2, gs3)
    ssem = (ss0, ss1, ss2, ss3)

    def _zb(t, _):
        zbuf[t, :] = zeros16
        return _
    lax.fori_loop(0, 64, _zb, None)

    def fire_gather(s, c):
        hoff = base_row + c * K
        pltpu.async_copy(sg.at[pl.ds(hoff, K)], sgb.at[s], gsem[s])
        pltpu.async_copy(feat.at[pl.ds(hoff, K), pl.ds(col0, HALF)],
                         fb.at[s], gsem[s])

    def wait_gather(s):
        pltpu.make_async_copy(sg.at[pl.ds(0, K)], sgb.at[s], gsem[s]).wait()
        pltpu.make_async_copy(feat.at[pl.ds(0, K), pl.ds(col0, HALF)],
                              fb.at[s], gsem[s]).wait()

    def fire_scatter(s):
        pltpu.async_copy(fb.at[s, pl.ds(0, 128), :],
                         spmem.at[ib.at[2 * s]], ssem[s], add=True)
        pltpu.async_copy(fb.at[s, pl.ds(128, 128), :],
                         spmem.at[ib.at[2 * s + 1]], ssem[s], add=True)

    def wait_scatter(s):
        pltpu.make_async_copy(fb.at[s], spmem.at[ib.at[2 * s]],
                              ssem[s]).wait()

    def build_idx(s, rbase):
        for q in range(0, K, L):
            seg = sgb[s, pl.ds(q, L)]
            d = seg - rbase
            ok = (d >= 0) & (d < RSIZE)
            dump = (tile * 256 + (q % 256)) + iota
            idx = jnp.where(ok, d + DUMP, dump)
            ib[2 * s + q // 128, pl.ds(q % 128, L)] = idx

    def sweep(p, _):
        rbase = p * RSIZE
        last = p == NSWEEP - 1
        sl = jnp.where(last, 4096, 6144)
        my0 = DUMP + tile * sl

        def _zero(k, _2):
            pltpu.sync_copy(zbuf, spmem.at[pl.ds(my0 + k * 64, 64), :])
            return _2
        lax.fori_loop(0, jnp.where(last, 64, 96), _zero, None)
        plsc.subcore_barrier()

        fire_gather(0, 0)
        fire_gather(1, 1)

        def quad(i, _2):
            for s in range(4):
                c = 4 * i + s
                wait_gather(s)
                build_idx(s, rbase)
                fire_scatter(s)
                t = (s + 2) % 4
                if s < 2:
                    @pl.when(i > 0)
                    def _w():
                        wait_scatter(t)
                else:
                    wait_scatter(t)
                fire_gather(t, c + 2)
            return _2
        lax.fori_loop(0, CHUNKS_MAIN // 4, quad, None)

        wait_gather(0)
        wait_gather(1)
        wait_scatter(2)
        wait_scatter(3)

        @pl.when(tile == NS - 1)
        def _extra():
            build_idx(0, rbase)
            fire_scatter(0)
            build_idx(1, rbase)
            fire_scatter(1)
            wait_scatter(0)
            wait_scatter(1)
            hoff = base_row + 246 * K
            pltpu.async_copy(sg.at[pl.ds(hoff, TAIL)],
                             sgb.at[2, pl.ds(0, TAIL)], gs2)
            pltpu.async_copy(feat.at[pl.ds(hoff, TAIL), pl.ds(col0, HALF)],
                             fb.at[2, pl.ds(0, TAIL), :], gs2)
            pltpu.make_async_copy(sg.at[pl.ds(0, TAIL)],
                                  sgb.at[2, pl.ds(0, TAIL)], gs2).wait()
            pltpu.make_async_copy(feat.at[pl.ds(0, TAIL), pl.ds(col0, HALF)],
                                  fb.at[2, pl.ds(0, TAIL), :], gs2).wait()
            for q in range(0, TAIL, L):
                seg = sgb[2, pl.ds(q, L)]
                d = seg - rbase
                ok = (d >= 0) & (d < RSIZE)
                dump = (tile * 256 + (q % 256)) + iota
                idx = jnp.where(ok, d + DUMP, dump)
                ibuf_t[pl.ds(q, L)] = idx
            pltpu.async_copy(fb.at[2, pl.ds(0, TAIL), :],
                             spmem.at[ibuf_t], ss2, add=True)
            pltpu.make_async_copy(fb.at[2, pl.ds(0, TAIL), :],
                                  spmem.at[ibuf_t], ss2).wait()

        plsc.subcore_barrier()

        def _copyout(k, _2):
            src0 = my0 + k * 256
            pltpu.sync_copy(spmem.at[pl.ds(src0, 256), :], obuf)

            def _scale(j, _3):
                for t in range(16):
                    row = j * 16 + t
                    obuf[row, :] = obuf[row, :] * SCALE
                return _3
            lax.fori_loop(0, 16, _scale, None)
            orow = rbase + tile * sl + k * 256
            pltpu.async_copy(
                obuf, out.at[pl.ds(orow, 256), pl.ds(col0, HALF)], osem).wait()
            return _2
        lax.fori_loop(0, jnp.where(last, 16, 24), _copyout, None)
        return _

    lax.fori_loop(0, NSWEEP, sweep, None)


def kernel(input_features, coords):
    seg = (((coords[:, 0] >> 1) << 12)
           + ((coords[:, 1] >> 1) << 6)
           + (coords[:, 2] >> 1)).astype(jnp.int32)

    fn = pl.kernel(
        _body,
        out_type=jax.ShapeDtypeStruct((S, C), jnp.float32),
        mesh=plsc.VectorSubcoreMesh(core_axis_name="c", subcore_axis_name="s"),
        compiler_params=pltpu.CompilerParams(use_tc_tiling_on_sc=False),
        scratch_types=[
            pltpu.VMEM((4, K), jnp.int32),
            pltpu.VMEM((4, K, HALF), jnp.float32),
            pltpu.VMEM((8, 128), jnp.int32),
            pltpu.VMEM((TAIL,), jnp.int32),
            pltpu.VMEM((256, HALF), jnp.float32),
            pltpu.VMEM((64, HALF), jnp.float32),
            pltpu.VMEM_SHARED((DUMP + RSIZE, HALF), jnp.float32),
            pltpu.SemaphoreType.DMA,
            pltpu.SemaphoreType.DMA,
            pltpu.SemaphoreType.DMA,
            pltpu.SemaphoreType.DMA,
            pltpu.SemaphoreType.DMA,
            pltpu.SemaphoreType.DMA,
            pltpu.SemaphoreType.DMA,
            pltpu.SemaphoreType.DMA,
            pltpu.SemaphoreType.DMA,
        ],
    )
    return fn(input_features, seg)


# speedup vs baseline: 1.4558x; 1.0915x over previous
import jax
import jax.numpy as jnp
from jax import lax
from jax.experimental import pallas as pl
from jax.experimental.pallas import tpu as pltpu
from jax.experimental.pallas import tpu_sc as plsc

N = 1_000_000
C = 32
HALF = 16
OUT_SIZE = 64
S = OUT_SIZE ** 3
SCALE = 0.125

NS = 16
L = 16

K = 512
ROWS_MAIN = 62_464
CHUNKS_MAIN = ROWS_MAIN // K
ROWS_LAST = N - (NS - 1) * ROWS_MAIN
TAIL = ROWS_LAST - 123 * K

DUMP = 1088
RSIZE = 90_112
NSWEEP = 3


def _body(feat, sg, out,
          sgb, fb, ib, ibuf_t, obuf, zbuf, spmem,
          gs0, gs1, gs2, gs3, ss0, ss1, ss2, ss3, osem):
    core = lax.axis_index("c")
    tile = lax.axis_index("s")
    base_row = tile * ROWS_MAIN
    col0 = core * HALF
    iota = lax.iota(jnp.int32, L)
    zeros16 = jnp.zeros((L,), jnp.float32)
    gsem = (gs0, gs1, gs2, gs3)
    ssem = (ss0, ss1, ss2, ss3)

    def _zb(t, _):
        zbuf[t, :] = zeros16
        return _
    lax.fori_loop(0, 32, _zb, None)

    def fire_gather(s, c):
        hoff = base_row + c * K
        pltpu.async_copy(sg.at[pl.ds(hoff, K)], sgb.at[s], gsem[s])
        pltpu.async_copy(feat.at[pl.ds(hoff, K), pl.ds(col0, HALF)],
                         fb.at[s], gsem[s])

    def wait_gather(s):
        pltpu.make_async_copy(sg.at[pl.ds(0, K)], sgb.at[s], gsem[s]).wait()
        pltpu.make_async_copy(feat.at[pl.ds(0, K), pl.ds(col0, HALF)],
                              fb.at[s], gsem[s]).wait()

    def fire_scatter(s):
        for j in range(4):
            pltpu.async_copy(fb.at[s, pl.ds(128 * j, 128), :],
                             spmem.at[ib.at[4 * s + j]], ssem[s], add=True)

    def wait_scatter(s):
        pltpu.make_async_copy(fb.at[s], spmem.at[ib.at[4 * s]],
                              ssem[s]).wait()

    def build_idx(s, rbase):
        for q in range(0, K, L):
            seg = sgb[s, pl.ds(q, L)]
            d = seg - rbase
            ok = (d >= 0) & (d < RSIZE)
            dump = (tile * 64 + (q % 48)) + iota
            idx = jnp.where(ok, d + DUMP, dump)
            ib[4 * s + q // 128, pl.ds(q % 128, L)] = idx

    def sweep(p, _):
        rbase = p * RSIZE
        last = p == NSWEEP - 1
        sl = jnp.where(last, 5120, 5632)
        my0 = DUMP + tile * sl

        def _zero(k, _2):
            for j in range(4):
                pltpu.async_copy(
                    zbuf, spmem.at[pl.ds(my0 + (4 * k + j) * 32, 32), :], osem)
            for j in range(4):
                pltpu.make_async_copy(
                    zbuf, spmem.at[pl.ds(my0, 32), :], osem).wait()
            return _2
        lax.fori_loop(0, jnp.where(last, 40, 44), _zero, None)
        plsc.subcore_barrier()

        fire_gather(0, 0)
        fire_gather(1, 1)

        def quad(i, _2):
            for s in range(4):
                c = 4 * i + s
                wait_gather(s)
                build_idx(s, rbase)
                fire_scatter(s)
                t = (s + 2) % 4
                if s < 2:
                    @pl.when(i > 0)
                    def _w():
                        wait_scatter(t)
                else:
                    wait_scatter(t)
                fire_gather(t, c + 2)
            return _2
        lax.fori_loop(0, 30, quad, None)

        wait_gather(0)
        build_idx(0, rbase)
        fire_scatter(0)
        wait_scatter(2)
        wait_gather(1)
        build_idx(1, rbase)
        fire_scatter(1)
        wait_scatter(3)
        wait_scatter(0)
        wait_scatter(1)

        @pl.when(tile == NS - 1)
        def _extra():
            fire_gather(2, 122)
            wait_gather(2)
            build_idx(2, rbase)
            fire_scatter(2)
            wait_scatter(2)
            hoff = base_row + 123 * K
            pltpu.async_copy(sg.at[pl.ds(hoff, TAIL)],
                             sgb.at[3, pl.ds(0, TAIL)], gs2)
            pltpu.async_copy(feat.at[pl.ds(hoff, TAIL), pl.ds(col0, HALF)],
                             fb.at[3, pl.ds(0, TAIL), :], gs2)
            pltpu.make_async_copy(sg.at[pl.ds(0, TAIL)],
                                  sgb.at[3, pl.ds(0, TAIL)], gs2).wait()
            pltpu.make_async_copy(feat.at[pl.ds(0, TAIL), pl.ds(col0, HALF)],
                                  fb.at[3, pl.ds(0, TAIL), :], gs2).wait()
            for q in range(0, TAIL, L):
                seg = sgb[3, pl.ds(q, L)]
                d = seg - rbase
                ok = (d >= 0) & (d < RSIZE)
                dump = (tile * 64 + (q % 48)) + iota
                idx = jnp.where(ok, d + DUMP, dump)
                ibuf_t[pl.ds(q, L)] = idx
            pltpu.async_copy(fb.at[3, pl.ds(0, TAIL), :],
                             spmem.at[ibuf_t], ss2, add=True)
            pltpu.make_async_copy(fb.at[3, pl.ds(0, TAIL), :],
                                  spmem.at[ibuf_t], ss2).wait()

        plsc.subcore_barrier()

        def _copyout(k, _2):
            src0 = my0 + k * 128
            pltpu.sync_copy(spmem.at[pl.ds(src0, 128), :], obuf)

            def _scale(j, _3):
                for t in range(16):
                    row = j * 16 + t
                    obuf[row, :] = obuf[row, :] * SCALE
                return _3
            lax.fori_loop(0, 8, _scale, None)
            orow = rbase + tile * sl + k * 128
            pltpu.async_copy(
                obuf, out.at[pl.ds(orow, 128), pl.ds(col0, HALF)], osem).wait()
            return _2
        lax.fori_loop(0, jnp.where(last, 40, 44), _copyout, None)
        return _

    lax.fori_loop(0, NSWEEP, sweep, None)


def kernel(input_features, coords):
    seg = (((coords[:, 0] >> 1) << 12)
           + ((coords[:, 1] >> 1) << 6)
           + (coords[:, 2] >> 1)).astype(jnp.int32)

    fn = pl.kernel(
        _body,
        out_type=jax.ShapeDtypeStruct((S, C), jnp.float32),
        mesh=plsc.VectorSubcoreMesh(core_axis_name="c", subcore_axis_name="s"),
        compiler_params=pltpu.CompilerParams(use_tc_tiling_on_sc=False),
        scratch_types=[
            pltpu.VMEM((4, K), jnp.int32),
            pltpu.VMEM((4, K, HALF), jnp.float32),
            pltpu.VMEM((16, 128), jnp.int32),
            pltpu.VMEM((TAIL,), jnp.int32),
            pltpu.VMEM((128, HALF), jnp.float32),
            pltpu.VMEM((32, HALF), jnp.float32),
            pltpu.VMEM_SHARED((DUMP + RSIZE, HALF), jnp.float32),
            pltpu.SemaphoreType.DMA,
            pltpu.SemaphoreType.DMA,
            pltpu.SemaphoreType.DMA,
            pltpu.SemaphoreType.DMA,
            pltpu.SemaphoreType.DMA,
            pltpu.SemaphoreType.DMA,
            pltpu.SemaphoreType.DMA,
            pltpu.SemaphoreType.DMA,
            pltpu.SemaphoreType.DMA,
        ],
    )
    return fn(input_features, seg)
